# baseline (speedup 1.0000x reference)
import functools

import jax
import jax.numpy as jnp
from jax import lax
from jax.experimental import pallas as pl
from jax.experimental.pallas import tpu as pltpu
from jax.experimental.pallas import tpu_sc as plsc

NN = 8
SENT = 8
HOPS = 4
B = 2048
D = 1024
S = 256
BT = 256
RT = 128
CAP = 3072
NTILE = CAP // RT
GROWS = CAP + RT
BIG = B + HOPS * CAP + BT
DUMPBLK = (B + HOPS * CAP) // RT
NW = 32
BPW = B // NW


def _proj_body(x_ref, Wp_ref, bp_ref, Wsym_ref, Wr_ref,
               big_ref, pi_ref, sym8_ref):
    z = jnp.dot(x_ref[...], Wp_ref[...],
                preferred_element_type=jnp.float32) + bp_ref[...]
    big_ref[...] = z
    for n in range(NN):
        sym8_ref[:, n, :] = jnp.tanh(
            jnp.dot(z, Wsym_ref[n], preferred_element_type=jnp.float32))
    logits = jnp.dot(z, Wr_ref[...], preferred_element_type=jnp.float32)
    cols = []
    for h in range(HOPS):
        sl = logits[:, h * (NN + 1):(h + 1) * (NN + 1)]
        best = sl[:, 0:1]
        idx = jnp.zeros((BT, 1), dtype=jnp.int32)
        for j in range(1, NN + 1):
            c = sl[:, j:j + 1] > best
            idx = jnp.where(c, j, idx)
            best = jnp.maximum(best, sl[:, j:j + 1])
        cols.append(idx)
    pi_ref[...] = jnp.concatenate(cols, axis=1)


def _projection(x, Wp, bp, Wsym, Wr):
    full = lambda t: (0, 0)
    return pl.pallas_call(
        _proj_body,
        grid=(B // BT,),
        in_specs=[
            pl.BlockSpec((BT, D), lambda t: (t, 0)),
            pl.BlockSpec((D, D), full),
            pl.BlockSpec((1, D), full),
            pl.BlockSpec((NN, D, S), lambda t: (0, 0, 0)),
            pl.BlockSpec((D, HOPS * (NN + 1)), full),
        ],
        out_specs=[
            pl.BlockSpec((BT, D), lambda t: (t, 0)),
            pl.BlockSpec((BT, HOPS), lambda t: (t, 0)),
            pl.BlockSpec((BT, NN, S), lambda t: (t, 0, 0)),
        ],
        out_shape=[
            jax.ShapeDtypeStruct((BIG, D), jnp.float32),
            jax.ShapeDtypeStruct((B, HOPS), jnp.int32),
            jax.ShapeDtypeStruct((B, NN, S), jnp.float32),
        ],
        compiler_params=pltpu.CompilerParams(
            dimension_semantics=("arbitrary",)),
    )(x, Wp, bp.reshape(1, D), Wsym, Wr)


def _onehots(pi):
    lanes = lax.broadcasted_iota(jnp.int32, (BT, 16), 1)
    ohs = []
    done = jnp.zeros((BT, 1), dtype=jnp.bool_)
    for h in range(HOPS):
        col = pi[:, h:h + 1]
        done = done | (col == SENT)
        e = jnp.where(done, SENT, col)
        ohs.append((lanes == e).astype(jnp.float32))
    return ohs


def _dispatch_body(pi_ref, dslot_ref, gsrc_ref, gsym_ref, te_ref,
                   counts_sc, carry_sc, off_sc):
    p = pl.program_id(0)
    t = pl.program_id(1)
    ohs = _onehots(pi_ref[...])

    @pl.when(p == 0)
    def _count():
        @pl.when(t == 0)
        def _z():
            counts_sc[...] = jnp.zeros((HOPS, 16), jnp.float32)
        for h in range(HOPS):
            counts_sc[h:h + 1, :] += jnp.sum(ohs[h], axis=0, keepdims=True)

    @pl.when(p == 1)
    def _place():
        @pl.when(t == 0)
        def _offsets():
            c = counts_sc[...]
            pc = jnp.ceil(c * (1.0 / RT)) * RT
            offs = []
            acc = jnp.zeros((HOPS, 1), jnp.float32)
            for e in range(16):
                if e < NN:
                    offs.append(acc)
                    acc = acc + pc[:, e:e + 1]
                else:
                    offs.append(jnp.full((HOPS, 1), float(CAP), jnp.float32))
            off = jnp.concatenate(offs, axis=1)
            off_sc[...] = off
            carry_sc[...] = jnp.zeros((HOPS, 16), jnp.float32)
            ts = lax.broadcasted_iota(
                jnp.int32, (HOPS, 32), 1).astype(jnp.float32) * RT
            te = jnp.full((HOPS, 32), SENT, jnp.int32)
            for e in range(NN):
                o = off[:, e:e + 1]
                m = (ts >= o) & (ts < o + pc[:, e:e + 1])
                te = jnp.where(m, e, te)
            te_ref[...] = te

        lt = (lax.broadcasted_iota(jnp.int32, (BT, BT), 1)
              < lax.broadcasted_iota(jnp.int32, (BT, BT), 0)
              ).astype(jnp.float32)
        dcols, gcols, scols = [], [], []
        rows = lax.broadcasted_iota(jnp.int32, (BT, 1), 0) + t * BT
        lanesf = lax.broadcasted_iota(
            jnp.int32, (BT, 16), 1).astype(jnp.float32)
        gix = rows
        for h in range(HOPS):
            oh = ohs[h]
            rank = jnp.dot(lt, oh, preferred_element_type=jnp.float32)
            sel = jnp.sum((rank + carry_sc[h:h + 1, :] + off_sc[h:h + 1, :])
                          * oh, axis=1, keepdims=True)
            carry_sc[h:h + 1, :] += jnp.sum(oh, axis=0, keepdims=True)
            ds = jnp.minimum(sel.astype(jnp.int32), CAP)
            e = jnp.sum(oh * lanesf, axis=1, keepdims=True).astype(jnp.int32)
            dcols.append(ds)
            scols.append(rows * NN + jnp.minimum(e, NN - 1))
            gcols.append(gix)
            gix = jnp.where(e < NN, B + h * CAP + ds, gix)
        gcols.append(gix)
        dslot_ref[...] = jnp.concatenate(dcols, axis=1)
        gsrc_ref[...] = jnp.concatenate(gcols, axis=1)
        gsym_ref[...] = jnp.concatenate(scols, axis=1)


def _dispatch(pi):
    return pl.pallas_call(
        _dispatch_body,
        grid=(2, B // BT),
        in_specs=[pl.BlockSpec((BT, HOPS), lambda p, t: (t, 0))],
        out_specs=[
            pl.BlockSpec((BT, HOPS), lambda p, t: (t, 0)),
            pl.BlockSpec((BT, HOPS + 1), lambda p, t: (t, 0)),
            pl.BlockSpec((BT, HOPS), lambda p, t: (t, 0)),
            pl.BlockSpec((HOPS, 32), lambda p, t: (0, 0)),
        ],
        out_shape=[
            jax.ShapeDtypeStruct((B, HOPS), jnp.int32),
            jax.ShapeDtypeStruct((B, HOPS + 1), jnp.int32),
            jax.ShapeDtypeStruct((B, HOPS), jnp.int32),
            jax.ShapeDtypeStruct((HOPS, 32), jnp.int32),
        ],
        scratch_shapes=[
            pltpu.VMEM((HOPS, 16), jnp.float32),
            pltpu.VMEM((HOPS, 16), jnp.float32),
            pltpu.VMEM((HOPS, 16), jnp.float32),
        ],
        compiler_params=pltpu.CompilerParams(
            dimension_semantics=("arbitrary", "arbitrary")),
    )(pi)


_MESH = None


def _mesh():
    global _MESH
    if _MESH is None:
        _MESH = plsc.VectorSubcoreMesh(core_axis_name="c", subcore_axis_name="s")
    return _MESH


def _wid():
    return lax.axis_index("s") * 2 + lax.axis_index("c")


def _sc_hop0(big, symflat, dslot, gsym):
    @functools.partial(
        pl.kernel, mesh=_mesh(),
        out_type=[jax.ShapeDtypeStruct((GROWS, D), jnp.float32),
                  jax.ShapeDtypeStruct((GROWS, S), jnp.float32)],
        scratch_types=[pltpu.VMEM((BPW,), jnp.int32),
                       pltpu.VMEM((BPW,), jnp.int32),
                       pltpu.VMEM((BPW, D), jnp.float32),
                       pltpu.VMEM((BPW, S), jnp.float32),
                       pltpu.SemaphoreType.DMA],
    )
    def k(big_hbm, sym_hbm, d_hbm, g_hbm, rows_out, symg_out,
          d_v, g_v, rows_v, symr_v, sem):
        base = _wid() * BPW
        pltpu.sync_copy(d_hbm.at[pl.ds(base, BPW)], d_v)
        pltpu.sync_copy(g_hbm.at[pl.ds(base, BPW)], g_v)
        pltpu.sync_copy(big_hbm.at[pl.ds(base, BPW)], rows_v)
        pltpu.async_copy(sym_hbm.at[g_v], symr_v, sem).wait()
        pltpu.async_copy(rows_v, rows_out.at[d_v], sem).wait()
        pltpu.async_copy(symr_v, symg_out.at[d_v], sem).wait()

    return k(big, symflat, dslot, gsym)


def _sc_sym123(symflat, dslots, gsyms):
    @functools.partial(
        pl.kernel, mesh=_mesh(),
        out_type=[jax.ShapeDtypeStruct((GROWS, S), jnp.float32)] * 3,
        scratch_types=[pltpu.VMEM((BPW,), jnp.int32),
                       pltpu.VMEM((BPW,), jnp.int32),
                       pltpu.VMEM((BPW, S), jnp.float32),
                       pltpu.SemaphoreType.DMA],
    )
    def k(sym_hbm, d1, d2, d3, g1, g2, g3, o1, o2, o3,
          d_v, g_v, symr_v, sem):
        base = _wid() * BPW
        for d_hbm, g_hbm, out in ((d1, g1, o1), (d2, g2, o2), (d3, g3, o3)):
            pltpu.sync_copy(d_hbm.at[pl.ds(base, BPW)], d_v)
            pltpu.sync_copy(g_hbm.at[pl.ds(base, BPW)], g_v)
            pltpu.async_copy(sym_hbm.at[g_v], symr_v, sem).wait()
            pltpu.async_copy(symr_v, out.at[d_v], sem).wait()

    return k(symflat, dslots[0], dslots[1], dslots[2],
             gsyms[0], gsyms[1], gsyms[2])


def _sc_rows(big, gsrc, dslot):
    @functools.partial(
        pl.kernel, mesh=_mesh(),
        out_type=jax.ShapeDtypeStruct((GROWS, D), jnp.float32),
        scratch_types=[pltpu.VMEM((BPW,), jnp.int32),
                       pltpu.VMEM((BPW,), jnp.int32),
                       pltpu.VMEM((BPW, D), jnp.float32),
                       pltpu.SemaphoreType.DMA],
    )
    def k(big_hbm, s_hbm, d_hbm, rows_out, s_v, d_v, rows_v, sem):
        base = _wid() * BPW
        pltpu.sync_copy(s_hbm.at[pl.ds(base, BPW)], s_v)
        pltpu.sync_copy(d_hbm.at[pl.ds(base, BPW)], d_v)
        pltpu.async_copy(big_hbm.at[s_v], rows_v, sem).wait()
        pltpu.async_copy(rows_v, rows_out.at[d_v], sem).wait()

    return k(big, gsrc, dslot)


def _sc_final(big, fin):
    @functools.partial(
        pl.kernel, mesh=_mesh(),
        out_type=jax.ShapeDtypeStruct((B, D), jnp.float32),
        scratch_types=[pltpu.VMEM((BPW,), jnp.int32),
                       pltpu.VMEM((BPW, D), jnp.float32),
                       pltpu.SemaphoreType.DMA],
    )
    def k(big_hbm, f_hbm, out_hbm, f_v, rows_v, sem):
        base = _wid() * BPW
        pltpu.sync_copy(f_hbm.at[pl.ds(base, BPW)], f_v)
        pltpu.async_copy(big_hbm.at[f_v], rows_v, sem).wait()
        pltpu.sync_copy(rows_v, out_hbm.at[pl.ds(base, BPW)])

    return k(big, fin)


def _mm_body(te_ref, rows_ref, symg_ref, W1_ref, Ws_ref, b1_ref, big_in_ref,
             big_ref):
    t = pl.program_id(0)
    e = te_ref[t]

    @pl.when(e < NN)
    def _compute():
        acc = jnp.dot(rows_ref[...], W1_ref[0],
                      preferred_element_type=jnp.float32)
        acc += jnp.dot(symg_ref[...], Ws_ref[0],
                       preferred_element_type=jnp.float32)
        big_ref[...] = jnp.maximum(acc + b1_ref[0], 0.0)


def _grouped_mm(h, te, rows_g, sym_g, W1, Ws, b1r, big):
    base = (B + h * CAP) // RT
    w_map = lambda t, te_ref: (jnp.minimum(te_ref[t], NN - 1), 0, 0)
    act = lambda t, te_ref: (jnp.where(te_ref[t] < NN, t, 0), 0)
    out_map = lambda t, te_ref: (
        jnp.where(te_ref[t] < NN, base + t, DUMPBLK), 0)
    grid_spec = pltpu.PrefetchScalarGridSpec(
        num_scalar_prefetch=1,
        grid=(NTILE,),
        in_specs=[
            pl.BlockSpec((RT, D), act),
            pl.BlockSpec((RT, S), act),
            pl.BlockSpec((1, D, D), w_map),
            pl.BlockSpec((1, S, D), w_map),
            pl.BlockSpec((1, 1, D), w_map),
            pl.BlockSpec(memory_space=pl.ANY),
        ],
        out_specs=pl.BlockSpec((RT, D), out_map),
    )
    return pl.pallas_call(
        _mm_body,
        grid_spec=grid_spec,
        out_shape=jax.ShapeDtypeStruct((BIG, D), jnp.float32),
        input_output_aliases={6: 0},
        compiler_params=pltpu.CompilerParams(
            dimension_semantics=("arbitrary",)),
    )(te, rows_g, sym_g, W1, Ws, b1r, big)


def kernel(x, max_ops, Wp, bp, Wsym, W1, Ws, b1, Wr):
    big, pi, sym = _projection(x, Wp, bp, Wsym, Wr)
    dslot, gsrc, gsym, te_all = _dispatch(pi)
    dT = dslot.T
    gT = gsrc.T
    sT = gsym.T
    symflat = sym.reshape(B * NN, S)
    b1r = b1.reshape(NN, 1, D)

    rows_g0, symg0 = _sc_hop0(big, symflat, dT[0], sT[0])
    symg1, symg2, symg3 = _sc_sym123(
        symflat, (dT[1], dT[2], dT[3]), (sT[1], sT[2], sT[3]))
    symgs = (symg0, symg1, symg2, symg3)

    rows_g = rows_g0
    for h in range(HOPS):
        big = _grouped_mm(h, te_all[h], rows_g, symgs[h], W1, Ws, b1r, big)
        if h + 1 < HOPS:
            rows_g = _sc_rows(big, gT[h + 1], dT[h + 1])
    out = _sc_final(big, gT[HOPS])
    return (out, pi, sym)


# speedup vs baseline: 1.6625x; 1.6625x over previous
import functools

import jax
import jax.numpy as jnp
from jax import lax
from jax.experimental import pallas as pl
from jax.experimental.pallas import tpu as pltpu
from jax.experimental.pallas import tpu_sc as plsc

NN = 8
SENT = 8
HOPS = 4
B = 2048
D = 1024
S = 256
BT = 256
RT = 128
CAP = 3072
NTILE = CAP // RT
GROWS = CAP + RT
NW = 32
BPW = B // NW


def _proj_body(x_ref, Wp_ref, bp_ref, Wsym_ref, Wr_ref,
               z_ref, pi_ref, sym8_ref):
    z = jnp.dot(x_ref[...], Wp_ref[...],
                preferred_element_type=jnp.float32) + bp_ref[...]
    z_ref[...] = z
    for n in range(NN):
        sym8_ref[:, n, :] = jnp.tanh(
            jnp.dot(z, Wsym_ref[n], preferred_element_type=jnp.float32))
    logits = jnp.dot(z, Wr_ref[...], preferred_element_type=jnp.float32)
    cols = []
    for h in range(HOPS):
        sl = logits[:, h * (NN + 1):(h + 1) * (NN + 1)]
        best = sl[:, 0:1]
        idx = jnp.zeros((BT, 1), dtype=jnp.int32)
        for j in range(1, NN + 1):
            c = sl[:, j:j + 1] > best
            idx = jnp.where(c, j, idx)
            best = jnp.maximum(best, sl[:, j:j + 1])
        cols.append(idx)
    pi_ref[...] = jnp.concatenate(cols, axis=1)


def _projection(x, Wp, bp, Wsym, Wr):
    full = lambda t: (0, 0)
    return pl.pallas_call(
        _proj_body,
        grid=(B // BT,),
        in_specs=[
            pl.BlockSpec((BT, D), lambda t: (t, 0)),
            pl.BlockSpec((D, D), full),
            pl.BlockSpec((1, D), full),
            pl.BlockSpec((NN, D, S), lambda t: (0, 0, 0)),
            pl.BlockSpec((D, HOPS * (NN + 1)), full),
        ],
        out_specs=[
            pl.BlockSpec((BT, D), lambda t: (t, 0)),
            pl.BlockSpec((BT, HOPS), lambda t: (t, 0)),
            pl.BlockSpec((BT, NN, S), lambda t: (t, 0, 0)),
        ],
        out_shape=[
            jax.ShapeDtypeStruct((B, D), jnp.float32),
            jax.ShapeDtypeStruct((B, HOPS), jnp.int32),
            jax.ShapeDtypeStruct((B, NN, S), jnp.float32),
        ],
        compiler_params=pltpu.CompilerParams(
            dimension_semantics=("arbitrary",)),
    )(x, Wp, bp.reshape(1, D), Wsym, Wr)


def _onehots(pi):
    lanes = lax.broadcasted_iota(jnp.int32, (BT, 16), 1)
    ohs = []
    done = jnp.zeros((BT, 1), dtype=jnp.bool_)
    for h in range(HOPS):
        col = pi[:, h:h + 1]
        done = done | (col == SENT)
        e = jnp.where(done, SENT, col)
        ohs.append((lanes == e).astype(jnp.float32))
    return ohs


def _dispatch_body(pi_ref, d_ref, g_ref, te_ref,
                   counts_sc, carry_sc, off_sc):
    p = pl.program_id(0)
    t = pl.program_id(1)
    ohs = _onehots(pi_ref[...])

    @pl.when(p == 0)
    def _count():
        @pl.when(t == 0)
        def _z():
            counts_sc[...] = jnp.zeros((HOPS, 16), jnp.float32)
        for h in range(HOPS):
            counts_sc[h:h + 1, :] += jnp.sum(ohs[h], axis=0, keepdims=True)

    @pl.when(p == 1)
    def _place():
        @pl.when(t == 0)
        def _offsets():
            c = counts_sc[...]
            pc = jnp.ceil(c * (1.0 / RT)) * RT
            offs = []
            acc = jnp.zeros((HOPS, 1), jnp.float32)
            for e in range(16):
                offs.append(acc)
                if e < NN:
                    acc = acc + pc[:, e:e + 1]
            off = jnp.concatenate(offs, axis=1)
            off_sc[...] = off
            carry_sc[...] = jnp.zeros((HOPS, 16), jnp.float32)
            ts = lax.broadcasted_iota(
                jnp.int32, (HOPS, 32), 1).astype(jnp.float32) * RT
            te = jnp.full((HOPS, 32), SENT, jnp.int32)
            for e in range(NN):
                o = off[:, e:e + 1]
                m = (ts >= o) & (ts < o + pc[:, e:e + 1])
                te = jnp.where(m, e, te)
            te_ref[...] = te

        lt = (lax.broadcasted_iota(jnp.int32, (BT, BT), 1)
              < lax.broadcasted_iota(jnp.int32, (BT, BT), 0)
              ).astype(jnp.float32)
        dcols, gcols = [], []
        rows = lax.broadcasted_iota(jnp.int32, (BT, 1), 0) + t * BT
        lanesf = lax.broadcasted_iota(
            jnp.int32, (BT, 16), 1).astype(jnp.float32)
        for h in range(HOPS):
            oh = ohs[h]
            rank = jnp.dot(lt, oh, preferred_element_type=jnp.float32)
            sel = jnp.sum((rank + carry_sc[h:h + 1, :] + off_sc[h:h + 1, :])
                          * oh, axis=1, keepdims=True)
            carry_sc[h:h + 1, :] += jnp.sum(oh, axis=0, keepdims=True)
            dcols.append(sel.astype(jnp.int32))
            e = jnp.sum(oh * lanesf, axis=1, keepdims=True).astype(jnp.int32)
            gcols.append(rows * NN + jnp.minimum(e, NN - 1))
        d_ref[...] = jnp.concatenate(dcols, axis=1)
        g_ref[...] = jnp.concatenate(gcols, axis=1)


def _dispatch(pi):
    return pl.pallas_call(
        _dispatch_body,
        grid=(2, B // BT),
        in_specs=[pl.BlockSpec((BT, HOPS), lambda p, t: (t, 0))],
        out_specs=[
            pl.BlockSpec((BT, HOPS), lambda p, t: (t, 0)),
            pl.BlockSpec((BT, HOPS), lambda p, t: (t, 0)),
            pl.BlockSpec((HOPS, 32), lambda p, t: (0, 0)),
        ],
        out_shape=[
            jax.ShapeDtypeStruct((B, HOPS), jnp.int32),
            jax.ShapeDtypeStruct((B, HOPS), jnp.int32),
            jax.ShapeDtypeStruct((HOPS, 32), jnp.int32),
        ],
        scratch_shapes=[
            pltpu.VMEM((HOPS, 16), jnp.float32),
            pltpu.VMEM((HOPS, 16), jnp.float32),
            pltpu.VMEM((HOPS, 16), jnp.float32),
        ],
        compiler_params=pltpu.CompilerParams(
            dimension_semantics=("arbitrary", "arbitrary")),
    )(pi)


_MESH = None


def _mesh():
    global _MESH
    if _MESH is None:
        _MESH = plsc.VectorSubcoreMesh(core_axis_name="c", subcore_axis_name="s")
    return _MESH


def _wid():
    return lax.axis_index("s") * 2 + lax.axis_index("c")


def _sc_scatter_first(z, symflat, d, g):
    @functools.partial(
        pl.kernel, mesh=_mesh(),
        out_type=[jax.ShapeDtypeStruct((GROWS, D), jnp.float32),
                  jax.ShapeDtypeStruct((CAP, S), jnp.float32)],
        scratch_types=[pltpu.VMEM((BPW,), jnp.int32),
                       pltpu.VMEM((BPW,), jnp.int32),
                       pltpu.VMEM((BPW, D), jnp.float32),
                       pltpu.VMEM((BPW, S), jnp.float32),
                       pltpu.SemaphoreType.DMA],
    )
    def k(z_hbm, sym_hbm, d_hbm, g_hbm, rows_out, symg_out,
          d_v, g_v, rows_v, symr_v, sem):
        base = _wid() * BPW
        pltpu.sync_copy(d_hbm.at[pl.ds(base, BPW)], d_v)
        pltpu.sync_copy(g_hbm.at[pl.ds(base, BPW)], g_v)
        pltpu.sync_copy(z_hbm.at[pl.ds(base, BPW)], rows_v)
        pltpu.async_copy(sym_hbm.at[g_v], symr_v, sem).wait()
        pltpu.async_copy(rows_v, rows_out.at[d_v], sem).wait()
        pltpu.async_copy(symr_v, symg_out.at[d_v], sem).wait()

    return k(z, symflat, d, g)


def _sc_sym123(symflat, ds, gs):
    @functools.partial(
        pl.kernel, mesh=_mesh(),
        out_type=[jax.ShapeDtypeStruct((CAP, S), jnp.float32)] * 3,
        scratch_types=[pltpu.VMEM((BPW,), jnp.int32),
                       pltpu.VMEM((BPW,), jnp.int32),
                       pltpu.VMEM((BPW, S), jnp.float32),
                       pltpu.SemaphoreType.DMA],
    )
    def k(sym_hbm, d1, d2, d3, g1, g2, g3, o1, o2, o3,
          d_v, g_v, symr_v, sem):
        base = _wid() * BPW
        for d_hbm, g_hbm, out in ((d1, g1, o1), (d2, g2, o2), (d3, g3, o3)):
            pltpu.sync_copy(d_hbm.at[pl.ds(base, BPW)], d_v)
            pltpu.sync_copy(g_hbm.at[pl.ds(base, BPW)], g_v)
            pltpu.async_copy(sym_hbm.at[g_v], symr_v, sem).wait()
            pltpu.async_copy(symr_v, out.at[d_v], sem).wait()

    return k(symflat, ds[0], ds[1], ds[2], gs[0], gs[1], gs[2])


def _sc_regroup(prev, dprev, d):
    @functools.partial(
        pl.kernel, mesh=_mesh(),
        out_type=jax.ShapeDtypeStruct((GROWS, D), jnp.float32),
        scratch_types=[pltpu.VMEM((BPW,), jnp.int32),
                       pltpu.VMEM((BPW,), jnp.int32),
                       pltpu.VMEM((BPW, D), jnp.float32),
                       pltpu.SemaphoreType.DMA],
    )
    def k(prev_hbm, dp_hbm, d_hbm, rows_out, dp_v, d_v, rows_v, sem):
        base = _wid() * BPW
        pltpu.sync_copy(dp_hbm.at[pl.ds(base, BPW)], dp_v)
        pltpu.sync_copy(d_hbm.at[pl.ds(base, BPW)], d_v)
        pltpu.async_copy(prev_hbm.at[dp_v], rows_v, sem).wait()
        pltpu.async_copy(rows_v, rows_out.at[d_v], sem).wait()

    return k(prev, dprev, d)


def _sc_gather_last(prev, dprev):
    @functools.partial(
        pl.kernel, mesh=_mesh(),
        out_type=jax.ShapeDtypeStruct((B, D), jnp.float32),
        scratch_types=[pltpu.VMEM((BPW,), jnp.int32),
                       pltpu.VMEM((BPW, D), jnp.float32),
                       pltpu.SemaphoreType.DMA],
    )
    def k(prev_hbm, dp_hbm, out_hbm, dp_v, rows_v, sem):
        base = _wid() * BPW
        pltpu.sync_copy(dp_hbm.at[pl.ds(base, BPW)], dp_v)
        pltpu.async_copy(prev_hbm.at[dp_v], rows_v, sem).wait()
        pltpu.sync_copy(rows_v, out_hbm.at[pl.ds(base, BPW)])

    return k(prev, dprev)


def _mm_body(te_ref, rows_ref, symg_ref, W1_ref, Ws_ref, b1_ref, out_ref):
    t = pl.program_id(0)
    e = te_ref[t]

    @pl.when(e < NN)
    def _compute():
        acc = jnp.dot(rows_ref[...], W1_ref[0],
                      preferred_element_type=jnp.float32)
        acc += jnp.dot(symg_ref[...], Ws_ref[0],
                       preferred_element_type=jnp.float32)
        out_ref[...] = jnp.maximum(acc + b1_ref[0], 0.0)


def _grouped_mm(te, rows_g, sym_g, W1, Ws, b1r):
    w_map = lambda t, te_ref: (jnp.minimum(te_ref[t], NN - 1), 0, 0)
    act = lambda t, te_ref: (jnp.where(te_ref[t] < NN, t, 0), 0)
    out_map = lambda t, te_ref: (jnp.where(te_ref[t] < NN, t, NTILE), 0)
    grid_spec = pltpu.PrefetchScalarGridSpec(
        num_scalar_prefetch=1,
        grid=(NTILE,),
        in_specs=[
            pl.BlockSpec((RT, D), act),
            pl.BlockSpec((RT, S), act),
            pl.BlockSpec((1, D, D), w_map),
            pl.BlockSpec((1, S, D), w_map),
            pl.BlockSpec((1, 1, D), w_map),
        ],
        out_specs=pl.BlockSpec((RT, D), out_map),
    )
    return pl.pallas_call(
        _mm_body,
        grid_spec=grid_spec,
        out_shape=jax.ShapeDtypeStruct((GROWS, D), jnp.float32),
        input_output_aliases={1: 0},
        compiler_params=pltpu.CompilerParams(
            dimension_semantics=("arbitrary",)),
    )(te, rows_g, sym_g, W1, Ws, b1r)


def kernel(x, max_ops, Wp, bp, Wsym, W1, Ws, b1, Wr):
    z, pi, sym = _projection(x, Wp, bp, Wsym, Wr)
    d_all, g_all, te_all = _dispatch(pi)
    dT = d_all.T
    gT = g_all.T
    symflat = sym.reshape(B * NN, S)
    b1r = b1.reshape(NN, 1, D)

    rows_g, symg0 = _sc_scatter_first(z, symflat, dT[0], gT[0])
    symg1, symg2, symg3 = _sc_sym123(
        symflat, (dT[1], dT[2], dT[3]), (gT[1], gT[2], gT[3]))
    symgs = (symg0, symg1, symg2, symg3)

    cur = None
    for h in range(HOPS):
        if h > 0:
            rows_g = _sc_regroup(cur, dT[h - 1], dT[h])
        cur = _grouped_mm(te_all[h], rows_g, symgs[h], W1, Ws, b1r)
    out = _sc_gather_last(cur, dT[HOPS - 1])
    return (out, pi, sym)
